# Initial kernel scaffold; baseline (speedup 1.0000x reference)
import jax
import jax.numpy as jnp
from jax.experimental import pallas as pl


def kernel(x, w_gate, W1, b1, W2, b2, gamma):
    raise NotImplementedError("write your pallas kernel here")


# speedup vs baseline: 3.8084x; 3.8084x over previous
import functools
import math

import jax
import jax.numpy as jnp
from jax.experimental import pallas as pl
from jax.experimental.pallas import tpu as pltpu

_K = 2


def _gelu_exact(v):
    return 0.5 * v * (1.0 + jax.lax.erf(v * (2.0 ** -0.5)))


def _cv_sq(v):
    m = jnp.mean(v)
    var = jnp.sum((v - m) ** 2) / (v.size - 1)
    return var / (m * m + 1e-10)


def _gating_body(x_ref, w_ref, gates_ref, gt_ref, loss_ref):
    logits = jnp.dot(x_ref[...], w_ref[...])
    p = jax.nn.softmax(logits, axis=1)
    T, E = p.shape
    lane = jax.lax.broadcasted_iota(jnp.int32, (T, E), 1)
    m1 = jnp.max(p, axis=1, keepdims=True)
    i1 = jnp.min(jnp.where(p == m1, lane, E), axis=1, keepdims=True)
    p2 = jnp.where(lane == i1, -1.0, p)
    m2 = jnp.max(p2, axis=1, keepdims=True)
    i2 = jnp.min(jnp.where(p2 == m2, lane, E), axis=1, keepdims=True)
    denom = m1 + m2 + 1e-6
    g1 = m1 / denom
    g2 = m2 / denom
    gates = jnp.where(lane == i1, g1, 0.0) + jnp.where(lane == i2, g2, 0.0)
    gates_ref[...] = gates
    gt_ref[...] = gates.T[:, None, :]
    importance = jnp.sum(gates, axis=0)
    load = jnp.sum((gates > 0).astype(jnp.float32), axis=0)
    loss_ref[...] = jnp.reshape(_cv_sq(importance) + _cv_sq(load), (1, 1))


def _gating(xf, w_gate):
    T, D = xf.shape
    E = w_gate.shape[1]
    return pl.pallas_call(
        _gating_body,
        out_shape=(
            jax.ShapeDtypeStruct((T, E), jnp.float32),
            jax.ShapeDtypeStruct((E, 1, T), jnp.float32),
            jax.ShapeDtypeStruct((1, 1), jnp.float32),
        ),
    )(xf, w_gate)


def _ffn_body(x_ref, gt_ref, w1_ref, b1_ref, w2_ref, b2_ref, out_ref, *, chunk):
    e = pl.program_id(0)

    @pl.when(e == 0)
    def _init():
        out_ref[...] = x_ref[...]

    T = x_ref.shape[0]
    w1 = w1_ref[0]
    w2 = w2_ref[0]
    b1 = b1_ref[0]
    b2 = b2_ref[0]
    for c in range(T // chunk):
        sl = pl.ds(c * chunk, chunk)
        xc = x_ref[sl, :]
        h = _gelu_exact(jnp.dot(xc, w1) + b1)
        o = jnp.dot(h, w2) + b2
        g = gt_ref[0, 0, sl][:, None]
        out_ref[sl, :] += g * o


def _moe_ffn(xf, gates_t, W1, b1, W2, b2, chunk=256):
    T, D = xf.shape
    E, _, H = W1.shape
    return pl.pallas_call(
        functools.partial(_ffn_body, chunk=chunk),
        grid=(E,),
        in_specs=[
            pl.BlockSpec((T, D), lambda e: (0, 0)),
            pl.BlockSpec((1, 1, T), lambda e: (e, 0, 0)),
            pl.BlockSpec((1, D, H), lambda e: (e, 0, 0)),
            pl.BlockSpec((1, 1, H), lambda e: (e, 0, 0)),
            pl.BlockSpec((1, H, D), lambda e: (e, 0, 0)),
            pl.BlockSpec((1, 1, D), lambda e: (e, 0, 0)),
        ],
        out_specs=pl.BlockSpec((T, D), lambda e: (0, 0)),
        out_shape=jax.ShapeDtypeStruct((T, D), jnp.float32),
    )(xf, gates_t, W1, b1.reshape(E, 1, H), W2, b2.reshape(E, 1, D))


def _norm_body(y_ref, gamma_ref, o_ref, *, sqrt_d):
    y = y_ref[...]
    n = jnp.sqrt(jnp.sum(y * y, axis=1, keepdims=True))
    scale = sqrt_d / jnp.maximum(n, 1e-12)
    o_ref[...] = _gelu_exact(y * scale * gamma_ref[0][None, :])


def _norm_gelu(y, gamma, block=256):
    T, D = y.shape
    return pl.pallas_call(
        functools.partial(_norm_body, sqrt_d=math.sqrt(D)),
        grid=(T // block,),
        in_specs=[
            pl.BlockSpec((block, D), lambda i: (i, 0)),
            pl.BlockSpec((1, D), lambda i: (0, 0)),
        ],
        out_specs=pl.BlockSpec((block, D), lambda i: (i, 0)),
        out_shape=jax.ShapeDtypeStruct((T, D), jnp.float32),
    )(y, gamma.reshape(1, D))


def kernel(x, w_gate, W1, b1, W2, b2, gamma):
    Bz, S_, D_ = x.shape
    xf = x.reshape(Bz * S_, D_)
    gates, gates_t, loss = _gating(xf, w_gate)
    y = _moe_ffn(xf, gates_t, W1, b1, W2, b2)
    out = _norm_gelu(y, gamma)
    del gates
    return out.reshape(Bz, S_, D_), loss[0, 0]
